# Initial kernel scaffold; baseline (speedup 1.0000x reference)
import jax
import jax.numpy as jnp
from jax.experimental import pallas as pl


def kernel(x, quant_weight, quant_absmax, quant_code):
    raise NotImplementedError("write your pallas kernel here")


# speedup vs baseline: 557.4367x; 557.4367x over previous
import functools

import jax
import jax.numpy as jnp
from jax import lax
from jax.experimental import pallas as pl
from jax.experimental.pallas import tpu as pltpu
from jax.experimental.pallas import tpu_sc as plsc

LANES = 16


def _build(V, D, B, A, mesh):
    NC = mesh.num_cores
    NS = mesh.num_subcores
    NW = NC * NS
    assert B % (NW * 128) == 0
    b_per_w = B // NW
    n_chunks = b_per_w // 128
    n_groups = b_per_w // LANES
    W = D // 4

    @functools.partial(
        pl.kernel,
        out_type=jax.ShapeDtypeStruct((B, D), jnp.float32),
        mesh=mesh,
        compiler_params=pltpu.CompilerParams(
            needs_layout_passes=False, use_tc_tiling_on_sc=False),
        scratch_types=[
            pltpu.VMEM((b_per_w,), jnp.int32),
            pltpu.VMEM((n_chunks, 128), jnp.int32),
            pltpu.VMEM((b_per_w, W), jnp.int32),
            pltpu.VMEM((A,), jnp.float32),
            pltpu.VMEM((256,), jnp.float32),
            pltpu.VMEM((b_per_w, D), jnp.float32),
            pltpu.SemaphoreType.DMA,
        ],
    )
    def deq_embed(x_hbm, qw_hbm, amax_hbm, code_hbm, out_hbm,
                  idx_v, idxg_v, rows_v, amax_v, code_v, out_v, sem):
        wid = lax.axis_index("s") * NC + lax.axis_index("c")
        base = wid * b_per_w

        for k in range(n_chunks):
            pltpu.sync_copy(x_hbm.at[pl.ds(base + k * 128, 128)], idxg_v.at[k])
        copies = [
            pltpu.async_copy(qw_hbm.at[idxg_v.at[k]],
                             rows_v.at[pl.ds(k * 128, 128)], sem)
            for k in range(n_chunks)
        ]
        pltpu.sync_copy(x_hbm.at[pl.ds(base, b_per_w)], idx_v)
        pltpu.sync_copy(amax_hbm, amax_v)
        pltpu.sync_copy(code_hbm, code_v)
        for cp in copies:
            cp.wait()

        iota = lax.broadcasted_iota(jnp.int32, (LANES,), 0)

        def group(g, _):
            row16 = g * LANES + iota
            idx16 = plsc.load_gather(idx_v, [row16])
            amax16 = plsc.load_gather(amax_v, [lax.shift_right_logical(idx16, 6)])
            for i in range(W):
                w = plsc.load_gather(rows_v, [row16, jnp.full((LANES,), i, jnp.int32)])
                for j in range(4):
                    q = lax.shift_right_logical(w, 8 * j) & 255 if j else w & 255
                    val = plsc.load_gather(code_v, [q]) * amax16
                    plsc.store_scatter(
                        out_v, [row16, jnp.full((LANES,), 4 * i + j, jnp.int32)], val)
            return 0

        lax.fori_loop(0, n_groups, group, 0)
        pltpu.sync_copy(out_v, out_hbm.at[pl.ds(base, b_per_w)])

    return deq_embed


def kernel(x, quant_weight, quant_absmax, quant_code):
    V, D = quant_weight.shape
    B = x.shape[0]
    A = quant_absmax.shape[0]
    qw_words = lax.bitcast_convert_type(
        quant_weight.reshape(V, D // 4, 4), jnp.int32)
    mesh = plsc.VectorSubcoreMesh(core_axis_name="c", subcore_axis_name="s")
    fn = _build(V, D, B, A, mesh)
    return fn(x, qw_words, quant_absmax, quant_code)


# speedup vs baseline: 1017.0099x; 1.8244x over previous
import functools

import jax
import jax.numpy as jnp
from jax import lax
from jax.experimental import pallas as pl
from jax.experimental.pallas import tpu as pltpu
from jax.experimental.pallas import tpu_sc as plsc

LANES = 16


def _build(V, D, B, A, mesh):
    NC = mesh.num_cores
    NS = mesh.num_subcores
    NW = NC * NS
    assert B % (NW * 128) == 0
    b_per_w = B // NW
    n_chunks = b_per_w // 128
    n_groups = b_per_w // LANES
    W = D // 4

    @functools.partial(
        pl.kernel,
        out_type=jax.ShapeDtypeStruct((B, D), jnp.float32),
        mesh=mesh,
        compiler_params=pltpu.CompilerParams(
            needs_layout_passes=False, use_tc_tiling_on_sc=False),
        scratch_types=[
            pltpu.VMEM((b_per_w,), jnp.int32),
            pltpu.VMEM((n_chunks, 128), jnp.int32),
            pltpu.VMEM((b_per_w, D), jnp.uint8),
            pltpu.VMEM((A,), jnp.float32),
            pltpu.VMEM((256,), jnp.float32),
            pltpu.VMEM((b_per_w, D), jnp.float32),
            pltpu.SemaphoreType.DMA,
        ],
    )
    def deq_embed(x_hbm, qw_hbm, amax_hbm, code_hbm, out_hbm,
                  idx_v, idxg_v, rows_v, amax_v, code_v, out_v, sem):
        wid = lax.axis_index("s") * NC + lax.axis_index("c")
        base = wid * b_per_w

        for k in range(n_chunks):
            pltpu.sync_copy(x_hbm.at[pl.ds(base + k * 128, 128)], idxg_v.at[k])
        copies = [
            pltpu.async_copy(qw_hbm.at[idxg_v.at[k]],
                             rows_v.at[pl.ds(k * 128, 128)], sem)
            for k in range(n_chunks)
        ]
        pltpu.sync_copy(x_hbm.at[pl.ds(base, b_per_w)], idx_v)
        pltpu.sync_copy(amax_hbm, amax_v)
        pltpu.sync_copy(code_hbm, code_v)
        for cp in copies:
            cp.wait()

        iota = lax.broadcasted_iota(jnp.int32, (LANES,), 0)

        def group(g, _):
            base_row = g * LANES
            idx16 = plsc.load_gather(idx_v, [base_row + iota])
            amax16 = plsc.load_gather(amax_v, [lax.shift_right_logical(idx16, 6)])
            for r in range(LANES):
                row = base_row + r
                w = plsc.bitcast(rows_v[row], jnp.int32)
                amax_r = lax.broadcast_in_dim(amax16[r], (LANES,), ())
                row_s = lax.broadcast_in_dim(row, (LANES,), ())
                for j in range(4):
                    q = lax.shift_right_logical(w, 8 * j) & 255 if j else w & 255
                    val = plsc.load_gather(code_v, [q]) * amax_r
                    plsc.store_scatter(out_v, [row_s, 4 * iota + j], val)
            return 0

        lax.fori_loop(0, n_groups, group, 0)
        pltpu.sync_copy(out_v, out_hbm.at[pl.ds(base, b_per_w)])

    return deq_embed


def kernel(x, quant_weight, quant_absmax, quant_code):
    V, D = quant_weight.shape
    B = x.shape[0]
    A = quant_absmax.shape[0]
    mesh = plsc.VectorSubcoreMesh(core_axis_name="c", subcore_axis_name="s")
    fn = _build(V, D, B, A, mesh)
    return fn(x, quant_weight, quant_absmax, quant_code)
